# Initial kernel scaffold; baseline (speedup 1.0000x reference)
import jax
import jax.numpy as jnp
from jax.experimental import pallas as pl


def kernel(samples):
    raise NotImplementedError("write your pallas kernel here")


# speedup vs baseline: 1903.0395x; 1903.0395x over previous
import functools

import jax
import jax.numpy as jnp
import numpy as np
from jax import lax
from jax.experimental import pallas as pl
from jax.experimental.pallas import tpu as pltpu
from jax.experimental.pallas import tpu_sc as plsc

_N = 8192
_B = 16384
_D = 128
_M = _B - _N

_NC = 2
_NS = 16
_NW = _NC * _NS
_BPW = _N // _NW
_CHUNK = 128
_K = _BPW // _CHUNK


def _rotl32(x: np.ndarray, d: int) -> np.ndarray:
    return ((x << np.uint32(d)) | (x >> np.uint32(32 - d))).astype(np.uint32)


def _threefry2x32(k0: int, k1: int, x0: np.ndarray, x1: np.ndarray):
    rot_a, rot_b = (13, 15, 26, 6), (17, 29, 16, 24)
    ks = [np.uint32(k0), np.uint32(k1),
          np.uint32(np.uint32(k0) ^ np.uint32(k1) ^ np.uint32(0x1BD11BDA))]
    x = [(x0 + ks[0]).astype(np.uint32), (x1 + ks[1]).astype(np.uint32)]

    def rounds(x, rots):
        for r in rots:
            x[0] = (x[0] + x[1]).astype(np.uint32)
            x[1] = _rotl32(x[1], r)
            x[1] = (x[0] ^ x[1]).astype(np.uint32)
        return x

    for i, rots in enumerate((rot_a, rot_b, rot_a, rot_b, rot_a)):
        x = rounds(x, rots)
        x[0] = (x[0] + ks[(i + 1) % 3]).astype(np.uint32)
        x[1] = (x[1] + ks[(i + 2) % 3] + np.uint32(i + 1)).astype(np.uint32)
    return x


def _uniform_key1(m: int) -> np.ndarray:
    i = np.arange(m, dtype=np.uint64)
    hi = (i >> np.uint64(32)).astype(np.uint32)
    lo = (i & np.uint64(0xFFFFFFFF)).astype(np.uint32)
    o0, o1 = _threefry2x32(0, 1, hi, lo)
    bits = o0 ^ o1
    return (((bits >> np.uint32(9)) | np.uint32(0x3F800000)).view(np.float32)
            - np.float32(1.0))


def _build_gather_map() -> np.ndarray:
    u = _uniform_key1(_M)
    sizes = (_N + np.arange(_M) + 1).astype(np.float32)
    idxs = np.floor(u * sizes).astype(np.int32)
    idxs = np.minimum(idxs, (sizes - 1).astype(np.int32))
    g = np.arange(_N, dtype=np.int32)
    for j in range(_M):
        if idxs[j] < _N:
            g[idxs[j]] = _N + j
    return g.reshape(_NW, _K, _CHUNK)


_GATHER_MAP = _build_gather_map()

_MESH = plsc.VectorSubcoreMesh(core_axis_name="c", subcore_axis_name="s")


@functools.partial(
    pl.kernel,
    mesh=_MESH,
    out_type=jax.ShapeDtypeStruct((_N, _D), jnp.float32),
    scratch_types=[
        pltpu.VMEM((_K, _CHUNK), jnp.int32),
        pltpu.VMEM((_BPW, _D), jnp.float32),
        pltpu.SemaphoreType.DMA,
    ],
)
def _gather_rows(samples_hbm, idx_hbm, out_hbm, idx_v, rows_v, sem):
    wid = lax.axis_index("s") * _NC + lax.axis_index("c")
    pltpu.sync_copy(idx_hbm.at[wid], idx_v)
    copies = []
    for j in range(_K):
        copies.append(
            pltpu.async_copy(
                samples_hbm.at[idx_v.at[j]],
                rows_v.at[pl.ds(j * _CHUNK, _CHUNK)],
                sem,
            )
        )
    for c in copies:
        c.wait()
    pltpu.sync_copy(rows_v, out_hbm.at[pl.ds(wid * _BPW, _BPW)])


def kernel(samples):
    return _gather_rows(samples, jnp.asarray(_GATHER_MAP))


# speedup vs baseline: 1906.9977x; 1.0021x over previous
import functools

import jax
import jax.numpy as jnp
import numpy as np
from jax import lax
from jax.experimental import pallas as pl
from jax.experimental.pallas import tpu as pltpu
from jax.experimental.pallas import tpu_sc as plsc

_N = 8192
_B = 16384
_D = 128
_M = _B - _N

_NC = 2
_NS = 16
_NW = _NC * _NS
_BPW = _N // _NW
_CHUNK = 128
_K = _BPW // _CHUNK


def _rotl32(x: np.ndarray, d: int) -> np.ndarray:
    return ((x << np.uint32(d)) | (x >> np.uint32(32 - d))).astype(np.uint32)


def _threefry2x32(k0: int, k1: int, x0: np.ndarray, x1: np.ndarray):
    rot_a, rot_b = (13, 15, 26, 6), (17, 29, 16, 24)
    ks = [np.uint32(k0), np.uint32(k1),
          np.uint32(np.uint32(k0) ^ np.uint32(k1) ^ np.uint32(0x1BD11BDA))]
    x = [(x0 + ks[0]).astype(np.uint32), (x1 + ks[1]).astype(np.uint32)]

    def rounds(x, rots):
        for r in rots:
            x[0] = (x[0] + x[1]).astype(np.uint32)
            x[1] = _rotl32(x[1], r)
            x[1] = (x[0] ^ x[1]).astype(np.uint32)
        return x

    for i, rots in enumerate((rot_a, rot_b, rot_a, rot_b, rot_a)):
        x = rounds(x, rots)
        x[0] = (x[0] + ks[(i + 1) % 3]).astype(np.uint32)
        x[1] = (x[1] + ks[(i + 2) % 3] + np.uint32(i + 1)).astype(np.uint32)
    return x


def _uniform_key1(m: int) -> np.ndarray:
    i = np.arange(m, dtype=np.uint64)
    hi = (i >> np.uint64(32)).astype(np.uint32)
    lo = (i & np.uint64(0xFFFFFFFF)).astype(np.uint32)
    o0, o1 = _threefry2x32(0, 1, hi, lo)
    bits = o0 ^ o1
    return (((bits >> np.uint32(9)) | np.uint32(0x3F800000)).view(np.float32)
            - np.float32(1.0))


def _build_gather_map() -> np.ndarray:
    u = _uniform_key1(_M)
    sizes = (_N + np.arange(_M) + 1).astype(np.float32)
    idxs = np.floor(u * sizes).astype(np.int32)
    idxs = np.minimum(idxs, (sizes - 1).astype(np.int32))
    g = np.arange(_N, dtype=np.int32)
    for j in range(_M):
        if idxs[j] < _N:
            g[idxs[j]] = _N + j
    return g.reshape(_NW, _K, _CHUNK)


_GATHER_MAP = _build_gather_map()

_MESH = plsc.VectorSubcoreMesh(core_axis_name="c", subcore_axis_name="s")


@functools.partial(
    pl.kernel,
    mesh=_MESH,
    out_type=jax.ShapeDtypeStruct((_N, _D), jnp.float32),
    scratch_types=[
        pltpu.VMEM((_K, _CHUNK), jnp.int32),
        pltpu.VMEM((_BPW, _D), jnp.float32),
        pltpu.SemaphoreType.DMA,
        pltpu.SemaphoreType.DMA,
    ],
)
def _gather_rows(samples_hbm, idx_hbm, out_hbm, idx_v, rows_v, gsem, wsem):
    wid = lax.axis_index("s") * _NC + lax.axis_index("c")
    base = wid * _BPW
    pltpu.sync_copy(idx_hbm.at[wid], idx_v)
    gathers = [
        pltpu.async_copy(
            samples_hbm.at[idx_v.at[j]],
            rows_v.at[pl.ds(j * _CHUNK, _CHUNK)],
            gsem,
        )
        for j in range(_K)
    ]
    writes = []
    for j in range(_K):
        gathers[j].wait()
        writes.append(
            pltpu.async_copy(
                rows_v.at[pl.ds(j * _CHUNK, _CHUNK)],
                out_hbm.at[pl.ds(base + j * _CHUNK, _CHUNK)],
                wsem,
            )
        )
    for w in writes:
        w.wait()


def kernel(samples):
    return _gather_rows(samples, jnp.asarray(_GATHER_MAP))
